# baseline (speedup 1.0000x reference)
import jax
import jax.numpy as jnp
from jax.experimental import pallas as pl
from jax.experimental.pallas import tpu as pltpu

_CHUNK_FRACS = (8, 8, 4, 2)
_NCHUNK = len(_CHUNK_FRACS)


def _chunks(n):
    sizes = [n // f for f in _CHUNK_FRACS]
    offs, o = [], 0
    for s in sizes:
        offs.append(o)
        o += s
    assert o == n
    return tuple(zip(offs, sizes))


def _body(idx_ref, arr_ref, out_ref, buf_ref, in_sems, out_sems):
    i = idx_ref[0]
    chunks = _chunks(out_ref.shape[1])
    ins = []
    for j, (o, c) in enumerate(chunks):
        cp = pltpu.make_async_copy(
            arr_ref.at[i, :, pl.ds(o, c)],
            buf_ref.at[:, pl.ds(o, c)],
            in_sems.at[j],
        )
        cp.start()
        ins.append(cp)
    outs = []
    for j, (o, c) in enumerate(chunks):
        ins[j].wait()
        cp = pltpu.make_async_copy(
            buf_ref.at[:, pl.ds(o, c)],
            out_ref.at[:, pl.ds(o, c)],
            out_sems.at[j],
        )
        cp.start()
        outs.append(cp)
    for cp in outs:
        cp.wait()


def kernel(x, arr, index):
    del x
    t, r, d = arr.shape
    idx = jnp.reshape(jnp.asarray(index, jnp.int32), (1,))
    arr_t = jnp.transpose(arr, (0, 2, 1))
    out_t = pl.pallas_call(
        _body,
        out_shape=jax.ShapeDtypeStruct((d, r), jnp.float32),
        in_specs=[
            pl.BlockSpec(memory_space=pltpu.MemorySpace.SMEM),
            pl.BlockSpec(memory_space=pl.ANY),
        ],
        out_specs=pl.BlockSpec(memory_space=pl.ANY),
        scratch_shapes=[
            pltpu.VMEM((d, r), jnp.float32),
            pltpu.SemaphoreType.DMA((_NCHUNK,)),
            pltpu.SemaphoreType.DMA((_NCHUNK,)),
        ],
    )(idx, arr_t)
    return out_t.T


# speedup vs baseline: 1.0199x; 1.0199x over previous
import jax
import jax.numpy as jnp
from jax.experimental import pallas as pl
from jax.experimental.pallas import tpu as pltpu

_NCHUNK = 4


def _body(idx_ref, arr_ref, out_ref, buf_ref, in_sems, out_sems):
    i = idx_ref[0]
    c = out_ref.shape[0] // _NCHUNK
    ins = []
    for j in range(_NCHUNK):
        cp = pltpu.make_async_copy(
            arr_ref.at[i, pl.ds(j * c, c), :],
            buf_ref.at[pl.ds(j * c, c), :],
            in_sems.at[j],
        )
        cp.start()
        ins.append(cp)
    outs = []
    for j in range(_NCHUNK):
        ins[j].wait()
        cp = pltpu.make_async_copy(
            buf_ref.at[pl.ds(j * c, c), :],
            out_ref.at[pl.ds(j * c, c), :],
            out_sems.at[j],
        )
        cp.start()
        outs.append(cp)
    for cp in outs:
        cp.wait()


def kernel(x, arr, index):
    del x
    t, r, d = arr.shape
    idx = jnp.reshape(jnp.asarray(index, jnp.int32), (1,))
    arr_t = jnp.transpose(arr, (0, 2, 1))
    out_t = pl.pallas_call(
        _body,
        out_shape=jax.ShapeDtypeStruct((d, r), jnp.float32),
        in_specs=[
            pl.BlockSpec(memory_space=pltpu.MemorySpace.SMEM),
            pl.BlockSpec(memory_space=pl.ANY),
        ],
        out_specs=pl.BlockSpec(memory_space=pl.ANY),
        scratch_shapes=[
            pltpu.VMEM((d, r), jnp.float32),
            pltpu.SemaphoreType.DMA((_NCHUNK,)),
            pltpu.SemaphoreType.DMA((_NCHUNK,)),
        ],
    )(idx, arr_t)
    return out_t.T
